# Initial kernel scaffold; baseline (speedup 1.0000x reference)
import jax
import jax.numpy as jnp
from jax.experimental import pallas as pl


def kernel(num_objs, token_pair_idx, token_pair_time, token_types, token_eidx, nfeats_lup, efeats_lup, bbox_feats, idx_in_lookup, n_id_lookup, attr_W, attr_b, bbox_W, bbox_b, time_freq, time_phase, n_id_W, n_id_b, type_emb):
    raise NotImplementedError("write your pallas kernel here")


# speedup vs baseline: 143.3001x; 143.3001x over previous
import functools

import jax
import jax.numpy as jnp
from jax import lax
from jax.experimental import pallas as pl
from jax.experimental.pallas import tpu as pltpu
from jax.experimental.pallas import tpu_sc as plsc

B, NC, L = 8, 8, 2048
MAX_OBJS, MAX_EDGES, NFEAT, NID = 128, 8192, 128, 32
OUT_DIM = 256
NTOK = B * NC * L

_SC_WORKERS = 32
_TOK_PER_W = NTOK // _SC_WORKERS
_CHUNK = 128
_NCHUNK = _TOK_PER_W // _CHUNK


def _sc_gather(gidx2d, efeats_flat):
    mesh = plsc.VectorSubcoreMesh(core_axis_name="c", subcore_axis_name="s")

    @functools.partial(
        pl.kernel,
        out_type=jax.ShapeDtypeStruct((NTOK, NFEAT), jnp.float32),
        mesh=mesh,
        scratch_types=[
            pltpu.VMEM((_NCHUNK, _CHUNK), jnp.int32),
            pltpu.VMEM((_CHUNK, NFEAT), jnp.float32),
            pltpu.SemaphoreType.DMA,
        ],
    )
    def k(gidx_hbm, tab_hbm, out_hbm, idx_v, rows_v, sem):
        wid = lax.axis_index("s") * 2 + lax.axis_index("c")
        base = wid * _TOK_PER_W
        pltpu.sync_copy(gidx_hbm.at[pl.ds(wid * _NCHUNK, _NCHUNK)], idx_v)

        def body(j, _):
            pltpu.async_copy(tab_hbm.at[idx_v.at[j]], rows_v, sem).wait()
            pltpu.sync_copy(rows_v, out_hbm.at[pl.ds(base + j * _CHUNK, _CHUNK)])
            return _

        lax.fori_loop(0, _NCHUNK, body, 0)

    return k(gidx2d, efeats_flat)


def _tc_body(tpi_ref, types_ref, time_ref, eg_ref, nf_ref, bbox_ref,
             nidx_ref, nlup_ref, attr_W_ref, attr_b_ref, bbox_W_ref,
             bbox_b_ref, freq_ref, phase_ref, nid_W_ref, nid_b_ref,
             temb_ref, out_ref):
    f32 = jnp.float32
    idx0 = tpi_ref[0, 0][:, 0:1]
    types = types_ref[0, 0]
    nonedge = (types == 0) | (types == 2)

    iota_o = lax.broadcasted_iota(jnp.int32, (L, MAX_OBJS), 1)
    oh_n = ((idx0 == iota_o) & nonedge).astype(f32)
    nf = jnp.dot(oh_n, nf_ref[0], preferred_element_type=f32)
    eg = jnp.where(nonedge, 0.0, eg_ref[0, 0])
    attr_feats = nf + eg
    attr = jnp.dot(attr_feats, attr_W_ref[...], preferred_element_type=f32)
    attr = attr + attr_b_ref[...]

    bbox = jnp.dot(bbox_ref[0, 0], bbox_W_ref[...], preferred_element_type=f32)
    bbox = bbox + bbox_b_ref[...]

    t0 = time_ref[0, 0][:, 0:1]
    t1 = time_ref[0, 0][:, 1:2]
    mx = jnp.max(jnp.maximum(t0, t1))
    h0 = jnp.cos((mx - t0) * freq_ref[...] + phase_ref[...])
    h1 = jnp.cos((mx - t1) * freq_ref[...] + phase_ref[...])

    ida = nidx_ref[0, 0][:, 0:1]
    idb = nidx_ref[0, 0][:, 1:2]
    iota_n = lax.broadcasted_iota(jnp.int32, (L, NID), 1)
    oh_a = (ida == iota_n).astype(f32)
    oh_b = (idb == iota_n).astype(f32)
    p_top = jnp.dot(nlup_ref[0, 0], nid_W_ref[0:NID, :],
                    preferred_element_type=f32)
    p_bot = jnp.dot(nlup_ref[0, 0], nid_W_ref[NID:2 * NID, :],
                    preferred_element_type=f32)
    nid = (jnp.dot(oh_a, p_top, preferred_element_type=f32)
           + jnp.dot(oh_b, p_bot, preferred_element_type=f32)
           + nid_b_ref[...])

    iota_t = lax.broadcasted_iota(jnp.int32, (L, 3), 1)
    oh_t = (types == iota_t).astype(f32)
    tfeat = jnp.dot(oh_t, temb_ref[...], preferred_element_type=f32)

    out = jnp.concatenate([attr, bbox, h0, h1, nid], axis=-1) + tfeat
    out_ref[0, 0] = out


def kernel(num_objs, token_pair_idx, token_pair_time, token_types, token_eidx,
           nfeats_lup, efeats_lup, bbox_feats, idx_in_lookup, n_id_lookup,
           attr_W, attr_b, bbox_W, bbox_b, time_freq, time_phase,
           n_id_W, n_id_b, type_emb):
    del num_objs
    gidx = (token_eidx.astype(jnp.int32)
            + (jnp.arange(B, dtype=jnp.int32) * MAX_EDGES)[:, None, None])
    gidx2d = gidx.reshape(NTOK // _CHUNK, _CHUNK)
    efeats_flat = efeats_lup.reshape(B * MAX_EDGES, NFEAT)

    egather = _sc_gather(gidx2d, efeats_flat).reshape(B, NC, L, NFEAT)

    types_r = token_types.astype(jnp.int32).reshape(B, NC, L, 1)
    tpi = token_pair_idx.astype(jnp.int32)
    nidx = idx_in_lookup.astype(jnp.int32).reshape(B, NC, L, 2)

    grid = (B, NC)
    bnc = lambda b, c: (b, c, 0, 0)
    full2 = lambda r, c: pl.BlockSpec((r, c), lambda b, n: (0, 0))

    out = pl.pallas_call(
        _tc_body,
        grid=grid,
        in_specs=[
            pl.BlockSpec((1, 1, L, 2), bnc),
            pl.BlockSpec((1, 1, L, 1), bnc),
            pl.BlockSpec((1, 1, L, 2), bnc),
            pl.BlockSpec((1, 1, L, NFEAT), bnc),
            pl.BlockSpec((1, MAX_OBJS, NFEAT), lambda b, n: (b, 0, 0)),
            pl.BlockSpec((1, 1, L, 8), bnc),
            pl.BlockSpec((1, 1, L, 2), bnc),
            pl.BlockSpec((1, 1, NID, NID), bnc),
            full2(NFEAT, 128),
            full2(1, 128),
            full2(8, 32),
            full2(1, 32),
            full2(1, 32),
            full2(1, 32),
            full2(2 * NID, 32),
            full2(1, 32),
            full2(3, OUT_DIM),
        ],
        out_specs=pl.BlockSpec((1, 1, L, OUT_DIM), bnc),
        out_shape=jax.ShapeDtypeStruct((B, NC, L, OUT_DIM), jnp.float32),
    )(
        tpi, types_r, token_pair_time, egather, nfeats_lup, bbox_feats,
        nidx, n_id_lookup, attr_W, attr_b.reshape(1, -1), bbox_W,
        bbox_b.reshape(1, -1), time_freq.reshape(1, -1),
        time_phase.reshape(1, -1), n_id_W, n_id_b.reshape(1, -1), type_emb,
    )
    return out


# speedup vs baseline: 238.0464x; 1.6612x over previous
import functools

import jax
import jax.numpy as jnp
from jax import lax
from jax.experimental import pallas as pl
from jax.experimental.pallas import tpu as pltpu
from jax.experimental.pallas import tpu_sc as plsc

B, NC, L = 8, 8, 2048
MAX_OBJS, MAX_EDGES, NFEAT, NID = 128, 8192, 128, 32
OUT_DIM = 256
NTOK = B * NC * L

_SC_WORKERS = 32
_TOK_PER_W = NTOK // _SC_WORKERS
_CHUNK = 128
_NCHUNK = _TOK_PER_W // _CHUNK


def _sc_gather(gidx2d, efeats_flat):
    mesh = plsc.VectorSubcoreMesh(core_axis_name="c", subcore_axis_name="s")

    @functools.partial(
        pl.kernel,
        out_type=jax.ShapeDtypeStruct((NTOK, NFEAT), jnp.float32),
        mesh=mesh,
        scratch_types=[
            pltpu.VMEM((_NCHUNK, _CHUNK), jnp.int32),
            pltpu.VMEM((_CHUNK, NFEAT), jnp.float32),
            pltpu.SemaphoreType.DMA,
        ],
    )
    def k(gidx_hbm, tab_hbm, out_hbm, idx_v, rows_v, sem):
        wid = lax.axis_index("s") * 2 + lax.axis_index("c")
        base = wid * _TOK_PER_W
        pltpu.sync_copy(gidx_hbm.at[pl.ds(wid * _NCHUNK, _NCHUNK)], idx_v)

        def body(j, _):
            pltpu.async_copy(tab_hbm.at[idx_v.at[j]], rows_v, sem).wait()
            pltpu.sync_copy(rows_v, out_hbm.at[pl.ds(base + j * _CHUNK, _CHUNK)])
            return _

        lax.fori_loop(0, _NCHUNK, body, 0)

    return k(gidx2d, efeats_flat)


_TWO_PI_HI = 6.28125
_TWO_PI_LO = 0.0019353071795864769
_INV_2PI = 0.15915494309189535


def _fast_cos(x):
    k = jnp.floor(x * _INV_2PI + 0.5)
    r = (x - k * _TWO_PI_HI) - k * _TWO_PI_LO
    r2 = r * r
    p = jnp.float32(1.7368827487e-09)
    p = p * r2 + jnp.float32(-2.7113293594e-07)
    p = p * r2 + jnp.float32(2.4773416502e-05)
    p = p * r2 + jnp.float32(-1.3887970073e-03)
    p = p * r2 + jnp.float32(4.1666524298e-02)
    p = p * r2 + jnp.float32(-4.9999991767e-01)
    p = p * r2 + jnp.float32(9.9999999227e-01)
    return p
def _tc_body(tpi_ref, types_ref, time_ref, time2d_ref, eg_ref, nf_ref,
             bbox_ref, nidx_ref, nlup_ref, attr_W_ref, attr_b_ref,
             bbox_W_ref, bbox_b_ref, fmat_ref, ff_ref, pp_ref, nid_W_ref,
             nid_b_ref, temb_ref, out_ref):
    f32 = jnp.float32
    idx0 = tpi_ref[0, 0][:, 0:1]
    types = types_ref[0, 0]
    nonedge = (types == 0) | (types == 2)

    iota_o = lax.broadcasted_iota(jnp.int32, (L, MAX_OBJS), 1)
    oh_n = ((idx0 == iota_o) & nonedge).astype(f32)
    nf = jnp.dot(oh_n, nf_ref[0], preferred_element_type=f32)
    eg = jnp.where(nonedge, 0.0, eg_ref[0, 0])
    attr_feats = nf + eg
    attr = jnp.dot(attr_feats, attr_W_ref[...], preferred_element_type=f32)
    attr = attr + attr_b_ref[...]

    bbox = jnp.dot(bbox_ref[0, 0], bbox_W_ref[...], preferred_element_type=f32)
    bbox = bbox + bbox_b_ref[...]

    mx = jnp.max(time2d_ref[0, 0])
    base = mx * ff_ref[...] + pp_ref[...]
    tf = jnp.dot(time_ref[0, 0], fmat_ref[...], preferred_element_type=f32)
    h01 = _fast_cos(base - tf)

    ida = nidx_ref[0, 0][:, 0:1]
    idb = nidx_ref[0, 0][:, 1:2]
    iota_n = lax.broadcasted_iota(jnp.int32, (L, NID), 1)
    oh_a = (ida == iota_n).astype(f32)
    oh_b = (idb == iota_n).astype(f32)
    p_top = jnp.dot(nlup_ref[0, 0], nid_W_ref[0:NID, :],
                    preferred_element_type=f32)
    p_bot = jnp.dot(nlup_ref[0, 0], nid_W_ref[NID:2 * NID, :],
                    preferred_element_type=f32)
    nid = (jnp.dot(oh_a, p_top, preferred_element_type=f32)
           + jnp.dot(oh_b, p_bot, preferred_element_type=f32)
           + nid_b_ref[...])

    iota_t = lax.broadcasted_iota(jnp.int32, (L, 3), 1)
    oh_t = (types == iota_t).astype(f32)
    tfeat = jnp.dot(oh_t, temb_ref[...], preferred_element_type=f32)

    out = jnp.concatenate([attr, bbox, h01, nid], axis=-1) + tfeat
    out_ref[0, 0] = out


def kernel(num_objs, token_pair_idx, token_pair_time, token_types, token_eidx,
           nfeats_lup, efeats_lup, bbox_feats, idx_in_lookup, n_id_lookup,
           attr_W, attr_b, bbox_W, bbox_b, time_freq, time_phase,
           n_id_W, n_id_b, type_emb):
    del num_objs
    gidx = (token_eidx.astype(jnp.int32)
            + (jnp.arange(B, dtype=jnp.int32) * MAX_EDGES)[:, None, None])
    gidx2d = gidx.reshape(NTOK // _CHUNK, _CHUNK)
    efeats_flat = efeats_lup.reshape(B * MAX_EDGES, NFEAT)

    egather = _sc_gather(gidx2d, efeats_flat).reshape(B, NC, L, NFEAT)

    half = time_freq.shape[0]
    z = jnp.zeros((half,), jnp.float32)
    fmat = jnp.stack([jnp.concatenate([time_freq, z]),
                      jnp.concatenate([z, time_freq])])
    ff = jnp.concatenate([time_freq, time_freq])
    pp = jnp.concatenate([time_phase, time_phase])

    types_r = token_types.astype(jnp.int32).reshape(B, NC, L, 1)
    tpi = token_pair_idx.astype(jnp.int32)
    nidx = idx_in_lookup.astype(jnp.int32).reshape(B, NC, L, 2)

    grid = (B, NC)
    bnc = lambda b, c: (b, c, 0, 0)
    full2 = lambda r, c: pl.BlockSpec((r, c), lambda b, n: (0, 0))

    out = pl.pallas_call(
        _tc_body,
        grid=grid,
        in_specs=[
            pl.BlockSpec((1, 1, L, 2), bnc),
            pl.BlockSpec((1, 1, L, 1), bnc),
            pl.BlockSpec((1, 1, L, 2), bnc),
            pl.BlockSpec((1, 1, 32, 128), bnc),
            pl.BlockSpec((1, 1, L, NFEAT), bnc),
            pl.BlockSpec((1, MAX_OBJS, NFEAT), lambda b, n: (b, 0, 0)),
            pl.BlockSpec((1, 1, L, 8), bnc),
            pl.BlockSpec((1, 1, L, 2), bnc),
            pl.BlockSpec((1, 1, NID, NID), bnc),
            full2(NFEAT, 128),
            full2(1, 128),
            full2(8, 32),
            full2(1, 32),
            full2(2, 64),
            full2(1, 64),
            full2(1, 64),
            full2(2 * NID, 32),
            full2(1, 32),
            full2(3, OUT_DIM),
        ],
        out_specs=pl.BlockSpec((1, 1, L, OUT_DIM), bnc),
        out_shape=jax.ShapeDtypeStruct((B, NC, L, OUT_DIM), jnp.float32),
    )(
        tpi, types_r, token_pair_time,
        token_pair_time.reshape(B, NC, 32, 128), egather, nfeats_lup, bbox_feats,
        nidx, n_id_lookup, attr_W, attr_b.reshape(1, -1), bbox_W,
        bbox_b.reshape(1, -1), fmat, ff.reshape(1, -1), pp.reshape(1, -1),
        n_id_W, n_id_b.reshape(1, -1), type_emb,
    )
    return out
